# baseline (speedup 1.0000x reference)
import functools

import jax
import jax.numpy as jnp
from jax import lax
from jax.experimental import pallas as pl
from jax.experimental.pallas import tpu as pltpu
from jax.experimental.pallas import tpu_sc as plsc

N = 10000
E = 320000
IN_DIM = 128
HID = 64

NC = 2
NS = 16
NW = NC * NS
NPAD = 10240
EPAD = 327680
EPW = EPAD // NW
C = 128
RPS = NPAD // NS

RB = 2000
GRID = N // RB


CHUNKS = EPAD // C
CPS = CHUNKS // NS
CPW0 = 160
CPW1 = CPS - CPW0
CPWMAX = max(CPW0, CPW1, 1)
K = 8


@functools.lru_cache(maxsize=None)
def _make_agg(D, dtype=jnp.float32):
    @functools.partial(
        pl.kernel,
        out_type=jax.ShapeDtypeStruct((NC, NPAD, D), dtype),
        mesh=plsc.VectorSubcoreMesh(core_axis_name="c", subcore_axis_name="s"),
        scratch_types=[
            pltpu.VMEM((CPWMAX, C), jnp.int32),
            pltpu.VMEM((CPWMAX, C), jnp.int32),
            pltpu.VMEM((2, K, C, D), dtype),
            pltpu.VMEM_SHARED((NPAD, D), dtype),
            pltpu.SemaphoreType.DMA,
            pltpu.SemaphoreType.DMA,
        ],
        compiler_params=pltpu.CompilerParams(use_tc_tiling_on_sc=False),
    )
    def agg(table, srcp, dstp, zeros, out, src_v, dst_v, rows_v, acc,
            gsem, ssem):
        cid = lax.axis_index("c")
        sid = lax.axis_index("s")
        r0 = sid * RPS
        pltpu.sync_copy(zeros, acc.at[pl.ds(r0, RPS)])
        plsc.subcore_barrier()

        def fire_gathers(b, buf):
            for j in range(K):
                pltpu.async_copy(
                    table.at[src_v.at[b * K + j]], rows_v.at[buf, j], gsem)

        def drain_gathers(buf):
            for j in range(K):
                pltpu.make_async_copy(
                    table.at[src_v.at[0]], rows_v.at[buf, j], gsem).wait()

        def fire_scatters(b, buf):
            for j in range(K):
                pltpu.async_copy(
                    rows_v.at[buf, j], acc.at[dst_v.at[b * K + j]], ssem,
                    add=True)

        def drain_scatters(buf):
            for j in range(K):
                pltpu.make_async_copy(
                    rows_v.at[buf, j], acc.at[dst_v.at[0]], ssem).wait()

        def run(base_chunk, cpw):
            if cpw == 0:
                return
            nbl = cpw // K
            pltpu.sync_copy(srcp.at[pl.ds(base_chunk, cpw)],
                            src_v.at[pl.ds(0, cpw)])
            pltpu.sync_copy(dstp.at[pl.ds(base_chunk, cpw)],
                            dst_v.at[pl.ds(0, cpw)])
            fire_gathers(0, 0)

            def body(b, carry):
                pb = lax.rem(b, 2)
                nb = lax.rem(b + 1, 2)
                drain_gathers(pb)
                fire_scatters(b, pb)
                pl.when(b >= 1)(lambda: drain_scatters(nb))
                pl.when(b + 1 < nbl)(lambda: fire_gathers(b + 1, nb))
                return carry

            lax.fori_loop(0, nbl, body, 0)
            drain_scatters(lax.rem(nbl - 1, 2))

        pl.when(cid == 0)(lambda: run(sid * CPS, CPW0))
        pl.when(cid == 1)(lambda: run(sid * CPS + CPW0, CPW1))
        plsc.subcore_barrier()
        pltpu.sync_copy(acc.at[pl.ds(r0, RPS)], out.at[cid, pl.ds(r0, RPS)])

    return agg


DEG_W = 8


@functools.lru_cache(maxsize=None)
def _make_deg():
    @functools.partial(
        pl.kernel,
        out_type=jax.ShapeDtypeStruct((NC, NPAD, 8), jnp.float32),
        mesh=plsc.VectorSubcoreMesh(core_axis_name="c", subcore_axis_name="s"),
        scratch_types=[
            pltpu.VMEM((CPWMAX, C), jnp.int32),
            pltpu.VMEM((C, 8), jnp.float32),
            pltpu.VMEM_SHARED((NPAD, 8), jnp.float32),
            pltpu.SemaphoreType.DMA,
        ],
        compiler_params=pltpu.CompilerParams(use_tc_tiling_on_sc=False),
    )
    def deg(ones, dstp, zeros, out, dst_v, ones_v, acc, ssem):
        cid = lax.axis_index("c")
        sid = lax.axis_index("s")
        r0 = sid * RPS
        pltpu.sync_copy(zeros, acc.at[pl.ds(r0, RPS)])
        pltpu.sync_copy(ones, ones_v)
        plsc.subcore_barrier()

        def run(base_chunk, cpw):
            if cpw == 0:
                return
            pltpu.sync_copy(dstp.at[pl.ds(base_chunk, cpw)],
                            dst_v.at[pl.ds(0, cpw)])

            def body(b, carry):
                pltpu.async_copy(ones_v, acc.at[dst_v.at[b]], ssem, add=True)
                pl.when(b >= DEG_W)(
                    lambda: pltpu.make_async_copy(
                        ones_v, acc.at[dst_v.at[0]], ssem).wait())
                return carry

            lax.fori_loop(0, cpw, body, 0)
            for _ in range(min(DEG_W, cpw)):
                pltpu.make_async_copy(ones_v, acc.at[dst_v.at[0]], ssem).wait()

        pl.when(cid == 0)(lambda: run(sid * CPS, CPW0))
        pl.when(cid == 1)(lambda: run(sid * CPS + CPW0, CPW1))
        plsc.subcore_barrier()
        pltpu.sync_copy(acc.at[pl.ds(r0, RPS)], out.at[cid, pl.ds(r0, RPS)])

    return deg


def _dis(deg_ref):
    deg = deg_ref[0, :, :1] + deg_ref[1, :, :1] + 1.0
    return lax.rsqrt(deg)


def _z1_body(x_ref, w_ref, deg_ref, o_ref):
    z = _dis(deg_ref) * jnp.dot(
        x_ref[...], w_ref[...], preferred_element_type=jnp.float32)
    o_ref[...] = z.astype(jnp.bfloat16)


def _z1_call(x, W1, deg2):
    return pl.pallas_call(
        _z1_body,
        grid=(GRID,),
        in_specs=[
            pl.BlockSpec((RB, IN_DIM), lambda i: (i, 0)),
            pl.BlockSpec((IN_DIM, HID), lambda i: (0, 0)),
            pl.BlockSpec((NC, RB, 8), lambda i: (0, i, 0)),
        ],
        out_specs=pl.BlockSpec((RB, HID), lambda i: (i, 0)),
        out_shape=jax.ShapeDtypeStruct((N, HID), jnp.bfloat16),
    )(x, W1, deg2)


def _mid_body(agg_ref, z1_ref, deg_ref, b1_ref, w2_ref, o_ref):
    dis = _dis(deg_ref)
    acc = (agg_ref[0].astype(jnp.float32) + agg_ref[1].astype(jnp.float32)
           + z1_ref[...].astype(jnp.float32))
    h = dis * acc + b1_ref[...]
    h = jnp.maximum(h, 0.0)
    o_ref[...] = dis * jnp.dot(h, w2_ref[...], preferred_element_type=jnp.float32)


def _mid_call(agg1, z1, deg2, b1r, w2p):
    return pl.pallas_call(
        _mid_body,
        grid=(GRID,),
        in_specs=[
            pl.BlockSpec((NC, RB, HID), lambda i: (0, i, 0)),
            pl.BlockSpec((RB, HID), lambda i: (i, 0)),
            pl.BlockSpec((NC, RB, 8), lambda i: (0, i, 0)),
            pl.BlockSpec((1, HID), lambda i: (0, 0)),
            pl.BlockSpec((HID, 8), lambda i: (0, 0)),
        ],
        out_specs=pl.BlockSpec((RB, 8), lambda i: (i, 0)),
        out_shape=jax.ShapeDtypeStruct((N, 8), jnp.float32),
    )(agg1, z1, deg2, b1r, w2p)


def _out_body(agg_ref, z2_ref, deg_ref, b2_ref, o_ref):
    dis = _dis(deg_ref)
    s = dis * (agg_ref[0] + agg_ref[1] + z2_ref[...]) + b2_ref[...]
    a = s[:, 0:1]
    b = s[:, 1:2]
    m = jnp.maximum(a, b)
    lse = m + jnp.log(jnp.exp(a - m) + jnp.exp(b - m))
    o_ref[...] = jnp.concatenate([a - lse, b - lse], axis=1)


def _out_call(agg2, z2p, deg2, b2p):
    return pl.pallas_call(
        _out_body,
        grid=(GRID,),
        in_specs=[
            pl.BlockSpec((NC, RB, 8), lambda i: (0, i, 0)),
            pl.BlockSpec((RB, 8), lambda i: (i, 0)),
            pl.BlockSpec((NC, RB, 8), lambda i: (0, i, 0)),
            pl.BlockSpec((1, 8), lambda i: (0, 0)),
        ],
        out_specs=pl.BlockSpec((RB, 2), lambda i: (i, 0)),
        out_shape=jax.ShapeDtypeStruct((N, 2), jnp.float32),
    )(agg2, z2p, deg2, b2p)


def kernel(x, edge_index, W1, b1, W2, b2):
    src = edge_index[0]
    dst = edge_index[1]
    pad_e = EPAD - E
    srcp = jnp.concatenate(
        [src, jnp.zeros((pad_e,), src.dtype)]).reshape(EPAD // C, C)
    dstp = jnp.concatenate(
        [dst, jnp.full((pad_e,), N, dst.dtype)]).reshape(EPAD // C, C)
    zeros8 = jnp.zeros((RPS, 8), jnp.float32)
    zeros64 = jnp.zeros((RPS, HID), jnp.bfloat16)
    ones_c8 = jnp.ones((C, 8), jnp.float32)

    deg2 = _make_deg()(ones_c8, dstp, zeros8)
    z1 = _z1_call(x, W1, deg2)
    agg1 = _make_agg(HID, jnp.bfloat16)(z1, srcp, dstp, zeros64)
    b1r = b1.reshape(1, HID)
    w2p = jnp.concatenate(
        [W2, jnp.zeros((HID, 8 - W2.shape[1]), W2.dtype)], axis=1)
    z2p = _mid_call(agg1, z1, deg2, b1r, w2p)
    agg2 = _make_agg(8)(z2p, srcp, dstp, zeros8)
    b2p = jnp.concatenate([b2, jnp.zeros((6,), b2.dtype)]).reshape(1, 8)
    return _out_call(agg2, z2p, deg2, b2p)


# speedup vs baseline: 1.0858x; 1.0858x over previous
import functools

import jax
import jax.numpy as jnp
from jax import lax
from jax.experimental import pallas as pl
from jax.experimental.pallas import tpu as pltpu
from jax.experimental.pallas import tpu_sc as plsc

N = 10000
E = 320000
IN_DIM = 128
HID = 64

NC = 2
NS = 16
NW = NC * NS
NPAD = 10240
EPAD = 327680
EPW = EPAD // NW
C = 128
RPS = NPAD // NS

RB = 2000
GRID = N // RB


CHUNKS = EPAD // C
CPS = CHUNKS // NS
CPW0 = 80
CPW1 = CPS - CPW0
CPWMAX = max(CPW0, CPW1, 1)
K = 8


@functools.lru_cache(maxsize=None)
def _make_agg(D, dtype=jnp.float32):
    @functools.partial(
        pl.kernel,
        out_type=jax.ShapeDtypeStruct((NC, NPAD, D), dtype),
        mesh=plsc.VectorSubcoreMesh(core_axis_name="c", subcore_axis_name="s"),
        scratch_types=[
            pltpu.VMEM((CPWMAX, C), jnp.int32),
            pltpu.VMEM((CPWMAX, C), jnp.int32),
            pltpu.VMEM((2, K, C, D), dtype),
            pltpu.VMEM_SHARED((NPAD, D), dtype),
            pltpu.SemaphoreType.DMA,
            pltpu.SemaphoreType.DMA,
        ],
        compiler_params=pltpu.CompilerParams(use_tc_tiling_on_sc=False),
    )
    def agg(table0, table1, srcp, dstp, zeros, out, src_v, dst_v, rows_v, acc,
            gsem, ssem):
        cid = lax.axis_index("c")
        sid = lax.axis_index("s")
        r0 = sid * RPS
        pltpu.sync_copy(zeros, acc.at[pl.ds(r0, RPS)])
        plsc.subcore_barrier()

        def fire_gathers(table, b, buf):
            for j in range(K):
                pltpu.async_copy(
                    table.at[src_v.at[b * K + j]], rows_v.at[buf, j], gsem)

        def drain_gathers(table, buf):
            for j in range(K):
                pltpu.make_async_copy(
                    table.at[src_v.at[0]], rows_v.at[buf, j], gsem).wait()

        def fire_scatters(b, buf):
            for j in range(K):
                pltpu.async_copy(
                    rows_v.at[buf, j], acc.at[dst_v.at[b * K + j]], ssem,
                    add=True)

        def drain_scatters(buf):
            for j in range(K):
                pltpu.make_async_copy(
                    rows_v.at[buf, j], acc.at[dst_v.at[0]], ssem).wait()

        def run(table, base_chunk, cpw):
            if cpw == 0:
                return
            nbl = cpw // K
            pltpu.sync_copy(srcp.at[pl.ds(base_chunk, cpw)],
                            src_v.at[pl.ds(0, cpw)])
            pltpu.sync_copy(dstp.at[pl.ds(base_chunk, cpw)],
                            dst_v.at[pl.ds(0, cpw)])
            fire_gathers(table, 0, 0)

            def body(b, carry):
                pb = lax.rem(b, 2)
                nb = lax.rem(b + 1, 2)
                drain_gathers(table, pb)
                fire_scatters(b, pb)
                pl.when(b >= 1)(lambda: drain_scatters(nb))
                pl.when(b + 1 < nbl)(lambda: fire_gathers(table, b + 1, nb))
                return carry

            lax.fori_loop(0, nbl, body, 0)
            drain_scatters(lax.rem(nbl - 1, 2))

        pl.when(cid == 0)(lambda: run(table0, sid * CPS, CPW0))
        pl.when(cid == 1)(lambda: run(table1, sid * CPS + CPW0, CPW1))
        plsc.subcore_barrier()
        pltpu.sync_copy(acc.at[pl.ds(r0, RPS)], out.at[cid, pl.ds(r0, RPS)])

    return agg


DEG_W = 8


@functools.lru_cache(maxsize=None)
def _make_deg():
    @functools.partial(
        pl.kernel,
        out_type=jax.ShapeDtypeStruct((NC, NPAD, 8), jnp.float32),
        mesh=plsc.VectorSubcoreMesh(core_axis_name="c", subcore_axis_name="s"),
        scratch_types=[
            pltpu.VMEM((CPWMAX, C), jnp.int32),
            pltpu.VMEM((C, 8), jnp.float32),
            pltpu.VMEM_SHARED((NPAD, 8), jnp.float32),
            pltpu.SemaphoreType.DMA,
        ],
        compiler_params=pltpu.CompilerParams(use_tc_tiling_on_sc=False),
    )
    def deg(ones, dstp, zeros, out, dst_v, ones_v, acc, ssem):
        cid = lax.axis_index("c")
        sid = lax.axis_index("s")
        r0 = sid * RPS
        pltpu.sync_copy(zeros, acc.at[pl.ds(r0, RPS)])
        pltpu.sync_copy(ones, ones_v)
        plsc.subcore_barrier()

        def run(base_chunk, cpw):
            if cpw == 0:
                return
            pltpu.sync_copy(dstp.at[pl.ds(base_chunk, cpw)],
                            dst_v.at[pl.ds(0, cpw)])

            def body(b, carry):
                pltpu.async_copy(ones_v, acc.at[dst_v.at[b]], ssem, add=True)
                pl.when(b >= DEG_W)(
                    lambda: pltpu.make_async_copy(
                        ones_v, acc.at[dst_v.at[0]], ssem).wait())
                return carry

            lax.fori_loop(0, cpw, body, 0)
            for _ in range(min(DEG_W, cpw)):
                pltpu.make_async_copy(ones_v, acc.at[dst_v.at[0]], ssem).wait()

        pl.when(cid == 0)(lambda: run(sid * CPS, CPW0))
        pl.when(cid == 1)(lambda: run(sid * CPS + CPW0, CPW1))
        plsc.subcore_barrier()
        pltpu.sync_copy(acc.at[pl.ds(r0, RPS)], out.at[cid, pl.ds(r0, RPS)])

    return deg


def _dis(deg_ref):
    deg = deg_ref[0, :, :1] + deg_ref[1, :, :1] + 1.0
    return lax.rsqrt(deg)


def _lin_body(x_ref, w_ref, o_ref):
    o_ref[...] = jnp.dot(
        x_ref[...], w_ref[...], preferred_element_type=jnp.float32)


def _lin_call(x, W1):
    return pl.pallas_call(
        _lin_body,
        grid=(GRID,),
        in_specs=[
            pl.BlockSpec((RB, IN_DIM), lambda i: (i, 0)),
            pl.BlockSpec((IN_DIM, HID), lambda i: (0, 0)),
        ],
        out_specs=pl.BlockSpec((RB, HID), lambda i: (i, 0)),
        out_shape=jax.ShapeDtypeStruct((N, HID), jnp.float32),
    )(x, W1)


def _z1_body(h_ref, deg_ref, oa_ref, ob_ref):
    z = (_dis(deg_ref) * h_ref[...]).astype(jnp.bfloat16)
    oa_ref[...] = z
    ob_ref[...] = z


def _z1_call(h1lin, deg2):
    return pl.pallas_call(
        _z1_body,
        grid=(GRID,),
        in_specs=[
            pl.BlockSpec((RB, HID), lambda i: (i, 0)),
            pl.BlockSpec((NC, RB, 8), lambda i: (0, i, 0)),
        ],
        out_specs=[
            pl.BlockSpec((RB, HID), lambda i: (i, 0)),
            pl.BlockSpec((RB, HID), lambda i: (i, 0)),
        ],
        out_shape=[
            jax.ShapeDtypeStruct((N, HID), jnp.bfloat16),
            jax.ShapeDtypeStruct((N, HID), jnp.bfloat16),
        ],
    )(h1lin, deg2)


def _mid_body(agg_ref, z1_ref, deg_ref, b1_ref, w2_ref, oa_ref, ob_ref):
    dis = _dis(deg_ref)
    acc = (agg_ref[0].astype(jnp.float32) + agg_ref[1].astype(jnp.float32)
           + z1_ref[...].astype(jnp.float32))
    h = dis * acc + b1_ref[...]
    h = jnp.maximum(h, 0.0)
    z2 = dis * jnp.dot(h, w2_ref[...], preferred_element_type=jnp.float32)
    oa_ref[...] = z2
    ob_ref[...] = z2


def _mid_call(agg1, z1, deg2, b1r, w2p):
    return pl.pallas_call(
        _mid_body,
        grid=(GRID,),
        in_specs=[
            pl.BlockSpec((NC, RB, HID), lambda i: (0, i, 0)),
            pl.BlockSpec((RB, HID), lambda i: (i, 0)),
            pl.BlockSpec((NC, RB, 8), lambda i: (0, i, 0)),
            pl.BlockSpec((1, HID), lambda i: (0, 0)),
            pl.BlockSpec((HID, 8), lambda i: (0, 0)),
        ],
        out_specs=[
            pl.BlockSpec((RB, 8), lambda i: (i, 0)),
            pl.BlockSpec((RB, 8), lambda i: (i, 0)),
        ],
        out_shape=[
            jax.ShapeDtypeStruct((N, 8), jnp.float32),
            jax.ShapeDtypeStruct((N, 8), jnp.float32),
        ],
    )(agg1, z1, deg2, b1r, w2p)


def _out_body(agg_ref, z2_ref, deg_ref, b2_ref, o_ref):
    dis = _dis(deg_ref)
    s = dis * (agg_ref[0] + agg_ref[1] + z2_ref[...]) + b2_ref[...]
    a = s[:, 0:1]
    b = s[:, 1:2]
    m = jnp.maximum(a, b)
    lse = m + jnp.log(jnp.exp(a - m) + jnp.exp(b - m))
    o_ref[...] = jnp.concatenate([a - lse, b - lse], axis=1)


def _out_call(agg2, z2p, deg2, b2p):
    return pl.pallas_call(
        _out_body,
        grid=(GRID,),
        in_specs=[
            pl.BlockSpec((NC, RB, 8), lambda i: (0, i, 0)),
            pl.BlockSpec((RB, 8), lambda i: (i, 0)),
            pl.BlockSpec((NC, RB, 8), lambda i: (0, i, 0)),
            pl.BlockSpec((1, 8), lambda i: (0, 0)),
        ],
        out_specs=pl.BlockSpec((RB, 2), lambda i: (i, 0)),
        out_shape=jax.ShapeDtypeStruct((N, 2), jnp.float32),
    )(agg2, z2p, deg2, b2p)


def kernel(x, edge_index, W1, b1, W2, b2):
    src = edge_index[0]
    dst = edge_index[1]
    pad_e = EPAD - E
    srcp = jnp.concatenate(
        [src, jnp.zeros((pad_e,), src.dtype)]).reshape(EPAD // C, C)
    dstp = jnp.concatenate(
        [dst, jnp.full((pad_e,), N, dst.dtype)]).reshape(EPAD // C, C)
    zeros8 = jnp.zeros((RPS, 8), jnp.float32)
    zeros64 = jnp.zeros((RPS, HID), jnp.bfloat16)
    ones_c8 = jnp.ones((C, 8), jnp.float32)

    deg2 = _make_deg()(ones_c8, dstp, zeros8)
    h1lin = _lin_call(x, W1)
    z1a, z1b = _z1_call(h1lin, deg2)
    agg1 = _make_agg(HID, jnp.bfloat16)(z1a, z1b, srcp, dstp, zeros64)
    b1r = b1.reshape(1, HID)
    w2p = jnp.concatenate(
        [W2, jnp.zeros((HID, 8 - W2.shape[1]), W2.dtype)], axis=1)
    z2a, z2b = _mid_call(agg1, z1a, deg2, b1r, w2p)
    agg2 = _make_agg(8)(z2a, z2b, srcp, dstp, zeros8)
    b2p = jnp.concatenate([b2, jnp.zeros((6,), b2.dtype)]).reshape(1, 8)
    return _out_call(agg2, z2a, deg2, b2p)


# speedup vs baseline: 1.0878x; 1.0019x over previous
import functools

import jax
import jax.numpy as jnp
from jax import lax
from jax.experimental import pallas as pl
from jax.experimental.pallas import tpu as pltpu
from jax.experimental.pallas import tpu_sc as plsc

N = 10000
E = 320000
IN_DIM = 128
HID = 64

NC = 2
NS = 16
NW = NC * NS
NPAD = 10240
EPAD = 327680
EPW = EPAD // NW
C = 128
RPS = NPAD // NS

RB = 2000
GRID = N // RB


CHUNKS = EPAD // C
CPS = CHUNKS // NS
CPW0 = 80
CPW1 = CPS - CPW0
CPWMAX = max(CPW0, CPW1, 1)
K = 8


@functools.lru_cache(maxsize=None)
def _make_agg(D, dtype=jnp.float32):
    @functools.partial(
        pl.kernel,
        out_type=jax.ShapeDtypeStruct((NC, NPAD, D), dtype),
        mesh=plsc.VectorSubcoreMesh(core_axis_name="c", subcore_axis_name="s"),
        scratch_types=[
            pltpu.VMEM((CPWMAX, C), jnp.int32),
            pltpu.VMEM((CPWMAX, C), jnp.int32),
            pltpu.VMEM((2, K, C, D), dtype),
            pltpu.VMEM_SHARED((NPAD, D), dtype),
            pltpu.SemaphoreType.DMA,
            pltpu.SemaphoreType.DMA,
        ],
        compiler_params=pltpu.CompilerParams(use_tc_tiling_on_sc=False),
    )
    def agg(table, srcp, dstp, zeros, out, src_v, dst_v, rows_v, acc,
            gsem, ssem):
        cid = lax.axis_index("c")
        sid = lax.axis_index("s")
        r0 = sid * RPS
        pltpu.sync_copy(zeros, acc.at[pl.ds(r0, RPS)])
        plsc.subcore_barrier()

        def fire_gathers(b, buf):
            for j in range(K):
                pltpu.async_copy(
                    table.at[src_v.at[b * K + j]], rows_v.at[buf, j], gsem)

        def drain_gathers(buf):
            for j in range(K):
                pltpu.make_async_copy(
                    table.at[src_v.at[0]], rows_v.at[buf, j], gsem).wait()

        def fire_scatters(b, buf):
            for j in range(K):
                pltpu.async_copy(
                    rows_v.at[buf, j], acc.at[dst_v.at[b * K + j]], ssem,
                    add=True)

        def drain_scatters(buf):
            for j in range(K):
                pltpu.make_async_copy(
                    rows_v.at[buf, j], acc.at[dst_v.at[0]], ssem).wait()

        def run(base_chunk, cpw):
            if cpw == 0:
                return
            nbl = cpw // K
            pltpu.sync_copy(srcp.at[pl.ds(base_chunk, cpw)],
                            src_v.at[pl.ds(0, cpw)])
            pltpu.sync_copy(dstp.at[pl.ds(base_chunk, cpw)],
                            dst_v.at[pl.ds(0, cpw)])
            fire_gathers(0, 0)

            def body(b, carry):
                pb = lax.rem(b, 2)
                nb = lax.rem(b + 1, 2)
                drain_gathers(pb)
                fire_scatters(b, pb)
                pl.when(b >= 1)(lambda: drain_scatters(nb))
                pl.when(b + 1 < nbl)(lambda: fire_gathers(b + 1, nb))
                return carry

            lax.fori_loop(0, nbl, body, 0)
            drain_scatters(lax.rem(nbl - 1, 2))

        pl.when(cid == 0)(lambda: run(sid * CPS, CPW0))
        pl.when(cid == 1)(lambda: run(sid * CPS + CPW0, CPW1))
        plsc.subcore_barrier()
        pltpu.sync_copy(acc.at[pl.ds(r0, RPS)], out.at[cid, pl.ds(r0, RPS)])

    return agg


DEG_W = 8


@functools.lru_cache(maxsize=None)
def _make_deg():
    @functools.partial(
        pl.kernel,
        out_type=jax.ShapeDtypeStruct((NC, NPAD, 8), jnp.float32),
        mesh=plsc.VectorSubcoreMesh(core_axis_name="c", subcore_axis_name="s"),
        scratch_types=[
            pltpu.VMEM((CPWMAX, C), jnp.int32),
            pltpu.VMEM((C, 8), jnp.float32),
            pltpu.VMEM_SHARED((NPAD, 8), jnp.float32),
            pltpu.SemaphoreType.DMA,
        ],
        compiler_params=pltpu.CompilerParams(use_tc_tiling_on_sc=False),
    )
    def deg(ones, dstp, zeros, out, dst_v, ones_v, acc, ssem):
        cid = lax.axis_index("c")
        sid = lax.axis_index("s")
        r0 = sid * RPS
        pltpu.sync_copy(zeros, acc.at[pl.ds(r0, RPS)])
        pltpu.sync_copy(ones, ones_v)
        plsc.subcore_barrier()

        def run(base_chunk, cpw):
            if cpw == 0:
                return
            pltpu.sync_copy(dstp.at[pl.ds(base_chunk, cpw)],
                            dst_v.at[pl.ds(0, cpw)])

            def body(b, carry):
                pltpu.async_copy(ones_v, acc.at[dst_v.at[b]], ssem, add=True)
                pl.when(b >= DEG_W)(
                    lambda: pltpu.make_async_copy(
                        ones_v, acc.at[dst_v.at[0]], ssem).wait())
                return carry

            lax.fori_loop(0, cpw, body, 0)
            for _ in range(min(DEG_W, cpw)):
                pltpu.make_async_copy(ones_v, acc.at[dst_v.at[0]], ssem).wait()

        pl.when(cid == 0)(lambda: run(sid * CPS, CPW0))
        pl.when(cid == 1)(lambda: run(sid * CPS + CPW0, CPW1))
        plsc.subcore_barrier()
        pltpu.sync_copy(acc.at[pl.ds(r0, RPS)], out.at[cid, pl.ds(r0, RPS)])

    return deg


def _dis(deg_ref):
    deg = deg_ref[0, :, :1] + deg_ref[1, :, :1] + 1.0
    return lax.rsqrt(deg)


def _lin_body(x_ref, w_ref, o_ref):
    o_ref[...] = jnp.dot(
        x_ref[...], w_ref[...], preferred_element_type=jnp.float32)


def _lin_call(x, W1):
    return pl.pallas_call(
        _lin_body,
        grid=(GRID,),
        in_specs=[
            pl.BlockSpec((RB, IN_DIM), lambda i: (i, 0)),
            pl.BlockSpec((IN_DIM, HID), lambda i: (0, 0)),
        ],
        out_specs=pl.BlockSpec((RB, HID), lambda i: (i, 0)),
        out_shape=jax.ShapeDtypeStruct((N, HID), jnp.float32),
    )(x, W1)


def _z1_body(h_ref, deg_ref, o_ref):
    o_ref[...] = (_dis(deg_ref) * h_ref[...]).astype(jnp.bfloat16)


def _z1_call(h1lin, deg2):
    return pl.pallas_call(
        _z1_body,
        grid=(GRID,),
        in_specs=[
            pl.BlockSpec((RB, HID), lambda i: (i, 0)),
            pl.BlockSpec((NC, RB, 8), lambda i: (0, i, 0)),
        ],
        out_specs=pl.BlockSpec((RB, HID), lambda i: (i, 0)),
        out_shape=jax.ShapeDtypeStruct((N, HID), jnp.bfloat16),
    )(h1lin, deg2)


def _mid_body(agg_ref, z1_ref, deg_ref, b1_ref, w2_ref, o_ref):
    dis = _dis(deg_ref)
    acc = (agg_ref[0].astype(jnp.float32) + agg_ref[1].astype(jnp.float32)
           + z1_ref[...].astype(jnp.float32))
    h = dis * acc + b1_ref[...]
    h = jnp.maximum(h, 0.0)
    o_ref[...] = dis * jnp.dot(h, w2_ref[...], preferred_element_type=jnp.float32)


def _mid_call(agg1, z1, deg2, b1r, w2p):
    return pl.pallas_call(
        _mid_body,
        grid=(GRID,),
        in_specs=[
            pl.BlockSpec((NC, RB, HID), lambda i: (0, i, 0)),
            pl.BlockSpec((RB, HID), lambda i: (i, 0)),
            pl.BlockSpec((NC, RB, 8), lambda i: (0, i, 0)),
            pl.BlockSpec((1, HID), lambda i: (0, 0)),
            pl.BlockSpec((HID, 8), lambda i: (0, 0)),
        ],
        out_specs=pl.BlockSpec((RB, 8), lambda i: (i, 0)),
        out_shape=jax.ShapeDtypeStruct((N, 8), jnp.float32),
    )(agg1, z1, deg2, b1r, w2p)


def _out_body(agg_ref, z2_ref, deg_ref, b2_ref, o_ref):
    dis = _dis(deg_ref)
    s = dis * (agg_ref[0] + agg_ref[1] + z2_ref[...]) + b2_ref[...]
    a = s[:, 0:1]
    b = s[:, 1:2]
    m = jnp.maximum(a, b)
    lse = m + jnp.log(jnp.exp(a - m) + jnp.exp(b - m))
    o_ref[...] = jnp.concatenate([a - lse, b - lse], axis=1)


def _out_call(agg2, z2p, deg2, b2p):
    return pl.pallas_call(
        _out_body,
        grid=(GRID,),
        in_specs=[
            pl.BlockSpec((NC, RB, 8), lambda i: (0, i, 0)),
            pl.BlockSpec((RB, 8), lambda i: (i, 0)),
            pl.BlockSpec((NC, RB, 8), lambda i: (0, i, 0)),
            pl.BlockSpec((1, 8), lambda i: (0, 0)),
        ],
        out_specs=pl.BlockSpec((RB, 2), lambda i: (i, 0)),
        out_shape=jax.ShapeDtypeStruct((N, 2), jnp.float32),
    )(agg2, z2p, deg2, b2p)


def kernel(x, edge_index, W1, b1, W2, b2):
    src = edge_index[0]
    dst = edge_index[1]
    pad_e = EPAD - E
    srcp = jnp.concatenate(
        [src, jnp.zeros((pad_e,), src.dtype)]).reshape(EPAD // C, C)
    dstp = jnp.concatenate(
        [dst, jnp.full((pad_e,), N, dst.dtype)]).reshape(EPAD // C, C)
    zeros8 = jnp.zeros((RPS, 8), jnp.float32)
    zeros64 = jnp.zeros((RPS, HID), jnp.bfloat16)
    ones_c8 = jnp.ones((C, 8), jnp.float32)

    deg2 = _make_deg()(ones_c8, dstp, zeros8)
    h1lin = _lin_call(x, W1)
    z1 = _z1_call(h1lin, deg2)
    agg1 = _make_agg(HID, jnp.bfloat16)(z1, srcp, dstp, zeros64)
    b1r = b1.reshape(1, HID)
    w2p = jnp.concatenate(
        [W2, jnp.zeros((HID, 8 - W2.shape[1]), W2.dtype)], axis=1)
    z2p = _mid_call(agg1, z1, deg2, b1r, w2p)
    agg2 = _make_agg(8)(z2p, srcp, dstp, zeros8)
    b2p = jnp.concatenate([b2, jnp.zeros((6,), b2.dtype)]).reshape(1, 8)
    return _out_call(agg2, z2p, deg2, b2p)


# speedup vs baseline: 1.1764x; 1.0814x over previous
import functools

import jax
import jax.numpy as jnp
from jax import lax
from jax.experimental import pallas as pl
from jax.experimental.pallas import tpu as pltpu
from jax.experimental.pallas import tpu_sc as plsc

N = 10000
E = 320000
IN_DIM = 128
HID = 64

NC = 2
NS = 16
NW = NC * NS
NPAD = 10240
EPAD = 327680
EPW = EPAD // NW
C = 128
RPS = NPAD // NS

RB = 2000
GRID = N // RB


CHUNKS = EPAD // C
CPS = CHUNKS // NS
CPW0 = 80
CPW1 = CPS - CPW0
CPWMAX = max(CPW0, CPW1, 1)
K = 8


@functools.lru_cache(maxsize=None)
def _make_agg(D, dtype=jnp.float32):
    @functools.partial(
        pl.kernel,
        out_type=jax.ShapeDtypeStruct((NC, NPAD, D), dtype),
        mesh=plsc.VectorSubcoreMesh(core_axis_name="c", subcore_axis_name="s"),
        scratch_types=[
            pltpu.VMEM((CPWMAX, C), jnp.int32),
            pltpu.VMEM((CPWMAX, C), jnp.int32),
            pltpu.VMEM((2, K, C, D), dtype),
            pltpu.VMEM_SHARED((NPAD, D), dtype),
            pltpu.SemaphoreType.DMA,
            pltpu.SemaphoreType.DMA,
        ],
        compiler_params=pltpu.CompilerParams(use_tc_tiling_on_sc=False),
    )
    def agg(table, srcp, dstp, zeros, out, src_v, dst_v, rows_v, acc,
            gsem, ssem):
        cid = lax.axis_index("c")
        sid = lax.axis_index("s")
        r0 = sid * RPS
        pltpu.sync_copy(zeros, acc.at[pl.ds(r0, RPS)])
        plsc.subcore_barrier()

        def fire_gathers(b, buf):
            for j in range(K):
                pltpu.async_copy(
                    table.at[src_v.at[b * K + j]], rows_v.at[buf, j], gsem)

        def drain_gathers(buf):
            for j in range(K):
                pltpu.make_async_copy(
                    table.at[src_v.at[0]], rows_v.at[buf, j], gsem).wait()

        def fire_scatters(b, buf):
            for j in range(K):
                pltpu.async_copy(
                    rows_v.at[buf, j], acc.at[dst_v.at[b * K + j]], ssem,
                    add=True)

        def drain_scatters(buf):
            for j in range(K):
                pltpu.make_async_copy(
                    rows_v.at[buf, j], acc.at[dst_v.at[0]], ssem).wait()

        def run(base_chunk, cpw):
            if cpw == 0:
                return
            nbl = cpw // K
            pltpu.sync_copy(srcp.at[pl.ds(base_chunk, cpw)],
                            src_v.at[pl.ds(0, cpw)])
            pltpu.sync_copy(dstp.at[pl.ds(base_chunk, cpw)],
                            dst_v.at[pl.ds(0, cpw)])
            fire_gathers(0, 0)

            def body(b, carry):
                pb = lax.rem(b, 2)
                nb = lax.rem(b + 1, 2)
                drain_gathers(pb)
                fire_scatters(b, pb)
                pl.when(b >= 1)(lambda: drain_scatters(nb))
                pl.when(b + 1 < nbl)(lambda: fire_gathers(b + 1, nb))
                return carry

            lax.fori_loop(0, nbl, body, 0)
            drain_scatters(lax.rem(nbl - 1, 2))

        pl.when(cid == 0)(lambda: run(sid * CPS, CPW0))
        pl.when(cid == 1)(lambda: run(sid * CPS + CPW0, CPW1))
        plsc.subcore_barrier()
        pltpu.sync_copy(acc.at[pl.ds(r0, RPS)], out.at[cid, pl.ds(r0, RPS)])

    return agg


DEG_W = 8


@functools.lru_cache(maxsize=None)
def _make_deg():
    @functools.partial(
        pl.kernel,
        out_type=jax.ShapeDtypeStruct((NC, NPAD, 8), jnp.float32),
        mesh=plsc.VectorSubcoreMesh(core_axis_name="c", subcore_axis_name="s"),
        scratch_types=[
            pltpu.VMEM((CPWMAX, C), jnp.int32),
            pltpu.VMEM((C, 8), jnp.float32),
            pltpu.VMEM_SHARED((NPAD, 8), jnp.float32),
            pltpu.SemaphoreType.DMA,
        ],
        compiler_params=pltpu.CompilerParams(use_tc_tiling_on_sc=False),
    )
    def deg(ones, dstp, zeros, out, dst_v, ones_v, acc, ssem):
        cid = lax.axis_index("c")
        sid = lax.axis_index("s")
        r0 = sid * RPS
        pltpu.sync_copy(zeros, acc.at[pl.ds(r0, RPS)])
        pltpu.sync_copy(ones, ones_v)
        plsc.subcore_barrier()

        def run(base_chunk, cpw):
            if cpw == 0:
                return
            pltpu.sync_copy(dstp.at[pl.ds(base_chunk, cpw)],
                            dst_v.at[pl.ds(0, cpw)])

            def body(b, carry):
                pltpu.async_copy(ones_v, acc.at[dst_v.at[b]], ssem, add=True)
                pl.when(b >= DEG_W)(
                    lambda: pltpu.make_async_copy(
                        ones_v, acc.at[dst_v.at[0]], ssem).wait())
                return carry

            lax.fori_loop(0, cpw, body, 0)
            for _ in range(min(DEG_W, cpw)):
                pltpu.make_async_copy(ones_v, acc.at[dst_v.at[0]], ssem).wait()

        pl.when(cid == 0)(lambda: run(sid * CPS, CPW0))
        pl.when(cid == 1)(lambda: run(sid * CPS + CPW0, CPW1))
        plsc.subcore_barrier()
        pltpu.sync_copy(acc.at[pl.ds(r0, RPS)], out.at[cid, pl.ds(r0, RPS)])

    return deg


def _dis(deg_ref):
    deg = deg_ref[0, :, :1] + deg_ref[1, :, :1] + 1.0
    return lax.rsqrt(deg)


def _z1_body(x_ref, w_ref, deg_ref, o_ref):
    z = _dis(deg_ref) * jnp.dot(
        x_ref[...], w_ref[...], preferred_element_type=jnp.float32)
    o_ref[...] = z.astype(jnp.bfloat16)


def _z1_call(x, W1, deg2):
    return pl.pallas_call(
        _z1_body,
        grid=(GRID,),
        in_specs=[
            pl.BlockSpec((RB, IN_DIM), lambda i: (i, 0)),
            pl.BlockSpec((IN_DIM, HID), lambda i: (0, 0)),
            pl.BlockSpec((NC, RB, 8), lambda i: (0, i, 0)),
        ],
        out_specs=pl.BlockSpec((RB, HID), lambda i: (i, 0)),
        out_shape=jax.ShapeDtypeStruct((N, HID), jnp.bfloat16),
    )(x, W1, deg2)


def _mid_body(agg_ref, z1_ref, deg_ref, b1_ref, w2_ref, o_ref):
    dis = _dis(deg_ref)
    acc = (agg_ref[0].astype(jnp.float32) + agg_ref[1].astype(jnp.float32)
           + z1_ref[...].astype(jnp.float32))
    h = dis * acc + b1_ref[...]
    h = jnp.maximum(h, 0.0)
    o_ref[...] = dis * jnp.dot(h, w2_ref[...], preferred_element_type=jnp.float32)


def _mid_call(agg1, z1, deg2, b1r, w2p):
    return pl.pallas_call(
        _mid_body,
        grid=(GRID,),
        in_specs=[
            pl.BlockSpec((NC, RB, HID), lambda i: (0, i, 0)),
            pl.BlockSpec((RB, HID), lambda i: (i, 0)),
            pl.BlockSpec((NC, RB, 8), lambda i: (0, i, 0)),
            pl.BlockSpec((1, HID), lambda i: (0, 0)),
            pl.BlockSpec((HID, 8), lambda i: (0, 0)),
        ],
        out_specs=pl.BlockSpec((RB, 8), lambda i: (i, 0)),
        out_shape=jax.ShapeDtypeStruct((N, 8), jnp.float32),
    )(agg1, z1, deg2, b1r, w2p)


def _out_body(agg_ref, z2_ref, deg_ref, b2_ref, o_ref):
    dis = _dis(deg_ref)
    s = dis * (agg_ref[0] + agg_ref[1] + z2_ref[...]) + b2_ref[...]
    a = s[:, 0:1]
    b = s[:, 1:2]
    m = jnp.maximum(a, b)
    lse = m + jnp.log(jnp.exp(a - m) + jnp.exp(b - m))
    o_ref[...] = jnp.concatenate([a - lse, b - lse], axis=1)


def _out_call(agg2, z2p, deg2, b2p):
    return pl.pallas_call(
        _out_body,
        grid=(GRID,),
        in_specs=[
            pl.BlockSpec((NC, RB, 8), lambda i: (0, i, 0)),
            pl.BlockSpec((RB, 8), lambda i: (i, 0)),
            pl.BlockSpec((NC, RB, 8), lambda i: (0, i, 0)),
            pl.BlockSpec((1, 8), lambda i: (0, 0)),
        ],
        out_specs=pl.BlockSpec((RB, 2), lambda i: (i, 0)),
        out_shape=jax.ShapeDtypeStruct((N, 2), jnp.float32),
    )(agg2, z2p, deg2, b2p)


def kernel(x, edge_index, W1, b1, W2, b2):
    src = edge_index[0]
    dst = edge_index[1]
    pad_e = EPAD - E
    srcp = jnp.concatenate(
        [src, jnp.zeros((pad_e,), src.dtype)]).reshape(EPAD // C, C)
    dstp = jnp.concatenate(
        [dst, jnp.full((pad_e,), N, dst.dtype)]).reshape(EPAD // C, C)
    zeros8 = jnp.zeros((RPS, 8), jnp.float32)
    zeros64 = jnp.zeros((RPS, HID), jnp.bfloat16)
    ones_c8 = jnp.ones((C, 8), jnp.float32)

    deg2 = _make_deg()(ones_c8, dstp, zeros8)
    z1 = _z1_call(x, W1, deg2)
    agg1 = _make_agg(HID, jnp.bfloat16)(z1, srcp, dstp, zeros64)
    b1r = b1.reshape(1, HID)
    w2p = jnp.concatenate(
        [W2, jnp.zeros((HID, 8 - W2.shape[1]), W2.dtype)], axis=1)
    z2p = _mid_call(agg1, z1, deg2, b1r, w2p)
    agg2 = _make_agg(8)(z2p, srcp, dstp, zeros8)
    b2p = jnp.concatenate([b2, jnp.zeros((6,), b2.dtype)]).reshape(1, 8)
    return _out_call(agg2, z2p, deg2, b2p)
